# Initial kernel scaffold; baseline (speedup 1.0000x reference)
import jax
import jax.numpy as jnp
from jax.experimental import pallas as pl


def kernel(x, edge_index, edge_label_index, l0_W1, l0_b1, l0_W2, l0_b2, l1_W1, l1_b1, l1_W2, l1_b2, weight1, weight2):
    raise NotImplementedError("write your pallas kernel here")


# speedup vs baseline: 6.9216x; 6.9216x over previous
import functools

import jax
import jax.numpy as jnp
from jax import lax
from jax.experimental import pallas as pl
from jax.experimental.pallas import tpu as pltpu
from jax.experimental.pallas import tpu_sc as plsc

_NS = 16
_NC = 2
_C = 128


def _sc_deg(dst3, zeros1d, n_nodes, r_nodes):
    ns, k, c = dst3.shape
    rpt = r_nodes // ns
    mesh = plsc.VectorSubcoreMesh(core_axis_name="c", subcore_axis_name="s")
    out = (jax.ShapeDtypeStruct((r_nodes,), jnp.float32),) * 2

    @functools.partial(
        pl.kernel, out_type=out, mesh=mesh,
        scratch_types=[
            pltpu.VMEM((k, c), jnp.int32),
            pltpu.VMEM((c,), jnp.float32),
            pltpu.VMEM((c,), jnp.float32),
            pltpu.VMEM_SHARED((r_nodes,), jnp.float32),
        ])
    def kern(dst_h, z_h, deg0_h, deg1_h, dstv, ones_v, zv, shared):
        ci = lax.axis_index("c")
        si = lax.axis_index("s")
        pltpu.sync_copy(z_h, zv)
        for m in range(rpt // c):
            pltpu.sync_copy(zv, shared.at[pl.ds(si * rpt + m * c, c)])
        for t in range(c // 16):
            ones_v[pl.ds(t * 16, 16)] = jnp.ones((16,), jnp.float32)
        pltpu.sync_copy(dst_h.at[si], dstv)
        plsc.subcore_barrier()
        kh = (k + 1) // 2
        lo = ci * kh
        hi = jnp.where(ci == 0, kh, k)

        def body(j, carry):
            pltpu.sync_copy(ones_v, shared.at[dstv.at[j]], add=True)
            return carry
        lax.fori_loop(lo, hi, body, 0)
        plsc.subcore_barrier()

        @pl.when(ci == 0)
        def _():
            pltpu.sync_copy(shared.at[pl.ds(si * rpt, rpt)],
                            deg0_h.at[pl.ds(si * rpt, rpt)])

        @pl.when(ci == 1)
        def _():
            pltpu.sync_copy(shared.at[pl.ds(si * rpt, rpt)],
                            deg1_h.at[pl.ds(si * rpt, rpt)])

    return kern(dst3, zeros1d)


def _sc_agg(hs0, hs1, src3, dst3, zeros2d, r_nodes):
    ns, k, c = src3.shape
    d2 = hs0.shape[1]
    rpt = r_nodes // ns
    mesh = plsc.VectorSubcoreMesh(core_axis_name="c", subcore_axis_name="s")
    out = (jax.ShapeDtypeStruct((r_nodes, d2), jnp.float32),) * 2

    @functools.partial(
        pl.kernel, out_type=out, mesh=mesh,
        scratch_types=[
            pltpu.VMEM((k, c), jnp.int32),
            pltpu.VMEM((k, c), jnp.int32),
            pltpu.VMEM((c, d2), jnp.float32),
            pltpu.VMEM_SHARED((r_nodes, d2), jnp.float32),
            pltpu.SemaphoreType.DMA,
        ])
    def kern(hs0_h, hs1_h, src_h, dst_h, z_h, agg0_h, agg1_h,
             srcv, dstv, rows, shared, sem):
        ci = lax.axis_index("c")
        si = lax.axis_index("s")
        pltpu.sync_copy(z_h, rows)
        for m in range(rpt // c):
            pltpu.sync_copy(rows, shared.at[pl.ds(si * rpt + m * c, c)])
        pltpu.sync_copy(src_h.at[si], srcv)
        pltpu.sync_copy(dst_h.at[si], dstv)
        plsc.subcore_barrier()

        def run(hs_h):
            def body(j, carry):
                pltpu.async_copy(hs_h.at[srcv.at[j]], rows, sem).wait()
                pltpu.sync_copy(rows, shared.at[dstv.at[j]], add=True)
                return carry
            lax.fori_loop(0, k, body, 0)

        @pl.when(ci == 0)
        def _():
            run(hs0_h)

        @pl.when(ci == 1)
        def _():
            run(hs1_h)

        plsc.subcore_barrier()

        @pl.when(ci == 0)
        def _():
            pltpu.sync_copy(shared.at[pl.ds(si * rpt, rpt)],
                            agg0_h.at[pl.ds(si * rpt, rpt)])

        @pl.when(ci == 1)
        def _():
            pltpu.sync_copy(shared.at[pl.ds(si * rpt, rpt)],
                            agg1_h.at[pl.ds(si * rpt, rpt)])

    return kern(hs0, hs1, src3, dst3, zeros2d)


def _sc_gather(p, eli3):
    nw, k, c = eli3.shape
    d = p.shape[1]
    tot = nw * k * c
    mesh = plsc.VectorSubcoreMesh(core_axis_name="c", subcore_axis_name="s")
    out = jax.ShapeDtypeStruct((tot, d), jnp.float32)

    @functools.partial(
        pl.kernel, out_type=out, mesh=mesh,
        scratch_types=[
            pltpu.VMEM((k, c), jnp.int32),
            pltpu.VMEM((c, d), jnp.float32),
            pltpu.SemaphoreType.DMA,
        ])
    def kern(p_h, eli_h, q_h, idxv, rows, sem):
        ci = lax.axis_index("c")
        si = lax.axis_index("s")
        w = si * _NC + ci
        pltpu.sync_copy(eli_h.at[w], idxv)

        def body(j, carry):
            pltpu.async_copy(p_h.at[idxv.at[j]], rows, sem).wait()
            pltpu.sync_copy(rows, q_h.at[pl.ds((w * k + j) * c, c)])
            return carry
        lax.fori_loop(0, k, body, 0)

    return kern(p, eli3)


def _linear_scale_body(x_ref, w_ref, b_ref, d0_ref, d1_ref, o0_ref, o1_ref):
    h = jnp.dot(x_ref[...], w_ref[...],
                preferred_element_type=jnp.float32) + b_ref[...]
    scale = lax.rsqrt(d0_ref[...] + d1_ref[...] + 1.0)
    hs = h * scale
    d2 = o0_ref.shape[1]
    o0_ref[...] = hs[:, :d2]
    o1_ref[...] = hs[:, d2:]


def _tc_linear_scale(x, w, b, deg0, deg1, bb):
    n, d = x.shape
    d2 = d // 2
    f = pl.pallas_call(
        _linear_scale_body,
        grid=(n // bb,),
        in_specs=[
            pl.BlockSpec((bb, d), lambda i: (i, 0)),
            pl.BlockSpec((d, d), lambda i: (0, 0)),
            pl.BlockSpec((1, d), lambda i: (0, 0)),
            pl.BlockSpec((bb, 1), lambda i: (i, 0)),
            pl.BlockSpec((bb, 1), lambda i: (i, 0)),
        ],
        out_specs=(pl.BlockSpec((bb, d2), lambda i: (i, 0)),
                   pl.BlockSpec((bb, d2), lambda i: (i, 0))),
        out_shape=(jax.ShapeDtypeStruct((n, d2), jnp.float32),) * 2,
    )
    return f(x, w, b.reshape(1, d), deg0, deg1)


def _aggpost_body(a0_ref, a1_ref, d0_ref, d1_ref, w_ref, b_ref, o_ref):
    a = jnp.concatenate([a0_ref[...], a1_ref[...]], axis=1)
    scale = lax.rsqrt(d0_ref[...] + d1_ref[...] + 1.0)
    y = jnp.maximum(a, 0.0) * scale
    o_ref[...] = jnp.maximum(
        jnp.dot(y, w_ref[...], preferred_element_type=jnp.float32)
        + b_ref[...], 0.0)


def _tc_aggpost(agg0, agg1, deg0, deg1, w, b, n, bb):
    d2 = agg0.shape[1]
    d = 2 * d2
    f = pl.pallas_call(
        _aggpost_body,
        grid=(n // bb,),
        in_specs=[
            pl.BlockSpec((bb, d2), lambda i: (i, 0)),
            pl.BlockSpec((bb, d2), lambda i: (i, 0)),
            pl.BlockSpec((bb, 1), lambda i: (i, 0)),
            pl.BlockSpec((bb, 1), lambda i: (i, 0)),
            pl.BlockSpec((d, d), lambda i: (0, 0)),
            pl.BlockSpec((1, d), lambda i: (0, 0)),
        ],
        out_specs=pl.BlockSpec((bb, d), lambda i: (i, 0)),
        out_shape=jax.ShapeDtypeStruct((n, d), jnp.float32),
    )
    return f(agg0, agg1, deg0, deg1, w, b.reshape(1, d))


def _final_body(h_ref, w1_ref, w2_ref, o_ref):
    h = h_ref[...]
    nrm = jnp.sqrt(jnp.sum(h * h, axis=1, keepdims=True))
    hn = h / jnp.maximum(nrm, 1e-12)
    dn = (((1,), (1,)), ((), ()))
    p1 = jnp.maximum(
        lax.dot_general(hn, w1_ref[...], dn,
                        preferred_element_type=jnp.float32), 0.0)
    p2 = lax.dot_general(p1, w2_ref[...], dn,
                         preferred_element_type=jnp.float32)
    o_ref[...] = jax.nn.sigmoid(p2)


def _tc_final(h, w1, w2, bb):
    n, d = h.shape
    f = pl.pallas_call(
        _final_body,
        grid=(n // bb,),
        in_specs=[
            pl.BlockSpec((bb, d), lambda i: (i, 0)),
            pl.BlockSpec((d, d), lambda i: (0, 0)),
            pl.BlockSpec((d, d), lambda i: (0, 0)),
        ],
        out_specs=pl.BlockSpec((bb, d), lambda i: (i, 0)),
        out_shape=jax.ShapeDtypeStruct((n, d), jnp.float32),
    )
    return f(h, w1, w2)


def _dot_body(q0_ref, q1_ref, o_ref):
    s = jnp.sum(q0_ref[...] * q1_ref[...], axis=1)
    o_ref[...] = s.reshape(o_ref.shape)


def _tc_dot(q, halfrows):
    tot, d = q.shape
    rb = 8 * _C
    nb = halfrows // rb
    f = pl.pallas_call(
        _dot_body,
        grid=(nb,),
        in_specs=[
            pl.BlockSpec((rb, d), lambda i: (i, 0)),
            pl.BlockSpec((rb, d), lambda i, _nb=nb: (i + _nb, 0)),
        ],
        out_specs=pl.BlockSpec((8, _C), lambda i: (i, 0)),
        out_shape=jax.ShapeDtypeStruct((nb * 8, _C), jnp.float32),
    )
    return f(q, q)


def kernel(x, edge_index, edge_label_index,
           l0_W1, l0_b1, l0_W2, l0_b2,
           l1_W1, l1_b1, l1_W2, l1_b2,
           weight1, weight2):
    n, d = x.shape
    e = edge_index.shape[1]
    el = edge_label_index.shape[1]
    bb = 2000
    assert n % bb == 0 and d == 256

    k_e = -(-e // (_NS * _C))
    e_pad = _NS * k_e * _C
    src3 = jnp.concatenate(
        [edge_index[0], jnp.zeros((e_pad - e,), jnp.int32)]).reshape(
            _NS, k_e, _C)
    dst3 = jnp.concatenate(
        [edge_index[1], jnp.full((e_pad - e,), n, jnp.int32)]).reshape(
            _NS, k_e, _C)

    rpt = -(-(n + 1) // (_NS * _C)) * _C
    r_nodes = rpt * _NS

    zeros1d = jnp.zeros((_C,), jnp.float32)
    zeros2d = jnp.zeros((_C, d // 2), jnp.float32)

    deg0, deg1 = _sc_deg(dst3, zeros1d, n, r_nodes)
    deg0 = deg0.reshape(r_nodes, 1)
    deg1 = deg1.reshape(r_nodes, 1)

    h = x
    for w1, b1, w2, b2 in ((l0_W1, l0_b1, l0_W2, l0_b2),
                           (l1_W1, l1_b1, l1_W2, l1_b2)):
        hs0, hs1 = _tc_linear_scale(h, w1, b1, deg0, deg1, bb)
        agg0, agg1 = _sc_agg(hs0, hs1, src3, dst3, zeros2d, r_nodes)
        h = _tc_aggpost(agg0, agg1, deg0, deg1, w2, b2, n, bb)

    p = _tc_final(h, weight1, weight2, bb)

    k_l = -(-2 * el // (_NS * _NC * _C))
    elp = k_l * _NS * _NC * _C // 2
    eli3 = jnp.pad(edge_label_index,
                   ((0, 0), (0, elp - el))).reshape(_NS * _NC, k_l, _C)
    q = _sc_gather(p, eli3)
    pred = _tc_dot(q, elp).reshape(elp)[:el]
    return pred
